# baseline (speedup 1.0000x reference)
import functools

import jax
import jax.numpy as jnp
from jax import lax
from jax.experimental import pallas as pl
from jax.experimental.pallas import tpu as pltpu
from jax.experimental.pallas import tpu_sc as plsc

NC = 2
NS = 16
L = 16
NW = NC * NS
BATCH = 16384
DIM = 64
PITCH = 128
VOCAB = 1000000
RB = 2048
BPW = BATCH // NW
CHUNK = 128
NCHUNK = BPW // CHUNK
UNROLL = L

_mesh = plsc.VectorSubcoreMesh(core_axis_name="c", subcore_axis_name="s")


def _rsqrt(s):
    s = jnp.maximum(s, jnp.float32(1e-24))
    i = lax.bitcast_convert_type(s, jnp.int32)
    i = jnp.int32(0x5F3759DF) - (i >> 1)
    y = lax.bitcast_convert_type(i, jnp.float32)
    for _ in range(3):
        y = y * (jnp.float32(1.5) - jnp.float32(0.5) * s * y * y)
    return y


def _repitch_body(e_in, r_in, e_out, r_out):
    e = e_in[...]
    r = r_in[...]
    e_out[...] = jnp.concatenate([e, e], axis=1)
    r_out[...] = jnp.concatenate([r, r], axis=1)


_repitch = pl.pallas_call(
    _repitch_body,
    grid=(VOCAB // RB,),
    in_specs=[pl.BlockSpec((RB, DIM), lambda i: (i, 0)),
              pl.BlockSpec((RB, DIM), lambda i: (i, 0))],
    out_specs=[pl.BlockSpec((RB, PITCH), lambda i: (i, 0)),
               pl.BlockSpec((RB, PITCH), lambda i: (i, 0))],
    out_shape=[jax.ShapeDtypeStruct((VOCAB, PITCH), jnp.float32),
               jax.ShapeDtypeStruct((VOCAB, PITCH), jnp.float32)],
)


@functools.partial(
    pl.kernel,
    out_type=jax.ShapeDtypeStruct((BATCH,), jnp.float32),
    mesh=_mesh,
    compiler_params=pltpu.CompilerParams(needs_layout_passes=False),
    scratch_types=[
        pltpu.VMEM((BPW,), jnp.int32),
        pltpu.VMEM((BPW,), jnp.int32),
        pltpu.VMEM((BPW,), jnp.int32),
        pltpu.VMEM((CHUNK, PITCH), jnp.float32),
        pltpu.VMEM((CHUNK, PITCH), jnp.float32),
        pltpu.VMEM((CHUNK, PITCH), jnp.float32),
        pltpu.VMEM((BPW,), jnp.float32),
        pltpu.SemaphoreType.DMA,
    ],
)
def _transe_kernel(hidx_hbm, ridx_hbm, tidx_hbm, ent_hbm, rel_hbm, out_hbm,
                   hidx_v, ridx_v, tidx_v, head_v, relrow_v, tail_v, out_v,
                   sem):
    wid = lax.axis_index("s") * NC + lax.axis_index("c")
    base = wid * BPW

    pltpu.sync_copy(hidx_hbm.at[pl.ds(base, BPW)], hidx_v)
    pltpu.sync_copy(ridx_hbm.at[pl.ds(base, BPW)], ridx_v)
    pltpu.sync_copy(tidx_hbm.at[pl.ds(base, BPW)], tidx_v)

    lanes = lax.iota(jnp.int32, L)

    for c in range(NCHUNK):
        isl = pl.ds(c * CHUNK, CHUNK)
        cp = [pltpu.async_copy(ent_hbm.at[hidx_v.at[isl]], head_v, sem),
              pltpu.async_copy(rel_hbm.at[ridx_v.at[isl]], relrow_v, sem),
              pltpu.async_copy(ent_hbm.at[tidx_v.at[isl]], tail_v, sem)]
        for x in cp:
            x.wait()

        def body(it, carry):
            vec = jnp.zeros((L,), jnp.float32)
            for u in range(UNROLL):
                i = it * UNROLL + u
                h = [head_v[i, pl.ds(L * k, L)] for k in range(DIM // L)]
                r = [relrow_v[i, pl.ds(L * k, L)] for k in range(DIM // L)]
                t = [tail_v[i, pl.ds(L * k, L)] for k in range(DIM // L)]
                hs = h[0] * h[0] + h[1] * h[1] + h[2] * h[2] + h[3] * h[3]
                ts = t[0] * t[0] + t[1] * t[1] + t[2] * t[2] + t[3] * t[3]
                ih = _rsqrt(jnp.sum(hs))
                itn = _rsqrt(jnp.sum(ts))
                acc = None
                for k in range(DIM // L):
                    term = jnp.abs(h[k] * ih + r[k] - t[k] * itn + 1e-6)
                    acc = term if acc is None else acc + term
                vec = jnp.where(lanes == u, jnp.sum(acc), vec)
            out_v[pl.ds(c * CHUNK + it * UNROLL, UNROLL)] = vec
            return carry

        lax.fori_loop(0, CHUNK // UNROLL, body, 0)

    pltpu.sync_copy(out_v, out_hbm.at[pl.ds(base, BPW)])


def kernel(triplet_idx, entity_emb, relation_emb):
    ent_p, rel_p = _repitch(entity_emb, relation_emb)
    return _transe_kernel(triplet_idx[:, 0], triplet_idx[:, 1],
                          triplet_idx[:, 2], ent_p, rel_p)
